# baseline (speedup 1.0000x reference)
import jax
import jax.numpy as jnp
from jax.experimental import pallas as pl
from jax.experimental.pallas import tpu as pltpu

_B = 64
_K = 16384
_N = 4096
_NB = 256
_KCH = 2048
_KPN = _K // _KCH
_TOT = (_N // _NB) * _KPN
_NBUF = 6


def _body(x_ref, b_ref, w_hbm, o_ref, xb_ref, buf_ref, bb_ref, sem_ref):
    def issue(c, slot):
        n = c // _KPN
        k = jax.lax.rem(c, _KPN)
        pltpu.make_async_copy(
            w_hbm.at[pl.ds(n * _NB, _NB), pl.ds(k * _KCH, _KCH)],
            buf_ref.at[slot],
            sem_ref.at[slot],
        ).start()

    def wait(c, slot):
        n = c // _KPN
        k = jax.lax.rem(c, _KPN)
        pltpu.make_async_copy(
            w_hbm.at[pl.ds(n * _NB, _NB), pl.ds(k * _KCH, _KCH)],
            buf_ref.at[slot],
            sem_ref.at[slot],
        ).wait()

    for j in range(_NBUF):
        issue(j, j)

    xb_ref[...] = x_ref[...].astype(jnp.bfloat16)

    wait(0, 0)
    bb_ref[0] = buf_ref[0].astype(jnp.bfloat16)
    issue(_NBUF, 0)

    def matmul(m, acc):
        k = jax.lax.rem(m, _KPN)
        wb = bb_ref[jax.lax.rem(m, 2)]
        xb = xb_ref[:, pl.ds(k * _KCH, _KCH)]
        part = jax.lax.dot_general(
            xb, wb, (((1,), (1,)), ((), ())),
            preferred_element_type=jnp.float32)
        acc = jnp.where(k == 0, part, acc + part)

        @pl.when(k == _KPN - 1)
        def _():
            n = m // _KPN
            o_ref[:, pl.ds(n * _NB, _NB)] = acc + b_ref[:, pl.ds(n * _NB, _NB)]

        return acc

    def step(c, acc):
        slot = jax.lax.rem(c, _NBUF)
        wait(c, slot)
        bb_ref[jax.lax.rem(c, 2)] = buf_ref[slot].astype(jnp.bfloat16)

        @pl.when(c + _NBUF < _TOT)
        def _():
            issue(c + _NBUF, slot)

        return matmul(c - 1, acc)

    acc = jax.lax.fori_loop(1, _TOT, step,
                            jnp.zeros((_B, _NB), jnp.float32))
    matmul(_TOT - 1, acc)


def kernel(input, weight, bias):
    bias2 = bias.reshape(1, _N)
    return pl.pallas_call(
        _body,
        in_specs=[
            pl.BlockSpec(memory_space=pltpu.MemorySpace.VMEM),
            pl.BlockSpec(memory_space=pltpu.MemorySpace.VMEM),
            pl.BlockSpec(memory_space=pltpu.MemorySpace.HBM),
        ],
        out_specs=pl.BlockSpec(memory_space=pltpu.MemorySpace.VMEM),
        out_shape=jax.ShapeDtypeStruct((_B, _N), jnp.float32),
        scratch_shapes=[
            pltpu.VMEM((_B, _K), jnp.bfloat16),
            pltpu.VMEM((_NBUF, _NB, _KCH), jnp.float32),
            pltpu.VMEM((2, _NB, _KCH), jnp.bfloat16),
            pltpu.SemaphoreType.DMA((_NBUF,)),
        ],
    )(input, bias2, weight)
